# baseline (speedup 1.0000x reference)
import functools

import jax
import jax.numpy as jnp
from jax import lax
from jax.experimental import pallas as pl
from jax.experimental.pallas import tpu as pltpu
from jax.experimental.pallas import tpu_sc as plsc

NUM_FIELDS = 26
VOCAB = 100000
EMBED_DIM = 16
BATCH = 16384

_NC = 2
_NS = 16
_NW = _NC * _NS

_LANES = 128
_FP = _LANES // EMBED_DIM
_COLS = 4
_NIDX = BATCH * _COLS * _FP
_B_PER_W = _NIDX // _NW
_CHUNK = 2048
_NCHUNK = _B_PER_W // _CHUNK


def _make_kernel():
  mesh = plsc.VectorSubcoreMesh(core_axis_name="c", subcore_axis_name="s")

  @functools.partial(
      pl.kernel,
      mesh=mesh,
      out_type=jax.ShapeDtypeStruct((_NIDX, EMBED_DIM), jnp.float32),
      compiler_params=pltpu.CompilerParams(use_tc_tiling_on_sc=False),
      scratch_types=[
          pltpu.VMEM((_B_PER_W,), jnp.int32),
          pltpu.VMEM((2, _CHUNK, EMBED_DIM), jnp.float32),
          pltpu.SemaphoreType.DMA,
          pltpu.SemaphoreType.DMA,
          pltpu.SemaphoreType.DMA,
          pltpu.SemaphoreType.DMA,
      ],
  )
  def emb_gather(idx_hbm, table_hbm, out_hbm, idx_v, rows_v, g0, g1, s0, s1):
    wid = lax.axis_index("s") * _NC + lax.axis_index("c")
    base = wid * _B_PER_W
    pltpu.sync_copy(idx_hbm.at[pl.ds(base, _B_PER_W)], idx_v)

    gsem = [g0, g1]
    ssem = [s0, s1]

    def gather(c):
      buf = c % 2
      return pltpu.async_copy(
          table_hbm.at[idx_v.at[pl.ds(c * _CHUNK, _CHUNK)]],
          rows_v.at[buf], gsem[buf])

    def put(c):
      buf = c % 2
      return pltpu.async_copy(
          rows_v.at[buf], out_hbm.at[pl.ds(base + c * _CHUNK, _CHUNK)],
          ssem[buf])

    gh = [None] * _NCHUNK
    ph = [None] * _NCHUNK
    gh[0] = gather(0)
    for c in range(_NCHUNK):
      if c + 1 < _NCHUNK:
        if c >= 1:
          ph[c - 1].wait()
        gh[c + 1] = gather(c + 1)
      gh[c].wait()
      ph[c] = put(c)
    ph[_NCHUNK - 2].wait()
    ph[_NCHUNK - 1].wait()

  return emb_gather


_EMB_GATHER = _make_kernel()


def kernel(features, tables):
  B, F = features.shape
  D = tables.shape[-1]
  rg = B // 8
  flat_tables = tables[:, :VOCAB, :].reshape(F * VOCAB, D)
  featpad = jnp.pad(features.astype(jnp.int32), ((0, 0), (0, _COLS * _FP - F)))
  vals4 = featpad.reshape(rg, 8, _COLS, _FP).transpose(0, 2, 1, 3)
  f4 = (jnp.arange(_COLS, dtype=jnp.int32)[None, :, None, None] * _FP
        + jnp.arange(_FP, dtype=jnp.int32)[None, None, None, :])
  offs4 = jnp.where(f4 < F, f4 * VOCAB, 0)
  idx = (vals4 + offs4).reshape(-1)
  out = _EMB_GATHER(idx, flat_tables)
  out = out.reshape(rg, _COLS, 8, _LANES).transpose(0, 2, 1, 3)
  return out.reshape(B, _COLS * _LANES)[:, :F * D]


# speedup vs baseline: 1.3973x; 1.3973x over previous
import functools

import jax
import jax.numpy as jnp
from jax import lax
from jax.experimental import pallas as pl
from jax.experimental.pallas import tpu as pltpu
from jax.experimental.pallas import tpu_sc as plsc

NUM_FIELDS = 26
VOCAB = 100000
EMBED_DIM = 16
BATCH = 16384

_NC = 2
_NS = 16
_NW = _NC * _NS

_LANES = 128
_FP = _LANES // EMBED_DIM
_COLS = 4
_N128 = BATCH * _COLS
_R_PER_W = _N128 // _NW
_C128 = 256
_NCHUNK = _R_PER_W // _C128
_IDX_PER_W = _R_PER_W * _FP


def _make_kernel():
  mesh = plsc.VectorSubcoreMesh(core_axis_name="c", subcore_axis_name="s")

  @functools.partial(
      pl.kernel,
      mesh=mesh,
      out_type=jax.ShapeDtypeStruct((_N128, _LANES), jnp.float32),
      compiler_params=pltpu.CompilerParams(use_tc_tiling_on_sc=False),
      scratch_types=[
          pltpu.VMEM((_IDX_PER_W,), jnp.int32),
          pltpu.VMEM((2, _FP, _C128, EMBED_DIM), jnp.float32),
          pltpu.SemaphoreType.DMA,
          pltpu.SemaphoreType.DMA,
          pltpu.SemaphoreType.DMA,
          pltpu.SemaphoreType.DMA,
      ],
  )
  def emb_gather(idx_hbm, table_hbm, out_hbm, idx_v, rows_v, g0, g1, s0, s1):
    wid = lax.axis_index("s") * _NC + lax.axis_index("c")
    base = wid * _IDX_PER_W
    base128 = wid * _R_PER_W
    pltpu.sync_copy(idx_hbm.at[pl.ds(base, _IDX_PER_W)], idx_v)

    gsem = [g0, g1]
    ssem = [s0, s1]

    def gather(c):
      buf = c % 2
      handles = []
      for j in range(_FP):
        handles.append(pltpu.async_copy(
            table_hbm.at[idx_v.at[pl.ds((c * _FP + j) * _C128, _C128)]],
            rows_v.at[buf, j], gsem[buf]))
      return handles

    def put(c):
      buf = c % 2
      handles = []
      for j in range(_FP):
        handles.append(pltpu.async_copy(
            rows_v.at[buf, j],
            out_hbm.at[pl.ds(base128 + c * _C128, _C128),
                       pl.ds(j * EMBED_DIM, EMBED_DIM)],
            ssem[buf]))
      return handles

    gh = [None] * _NCHUNK
    ph = [None] * _NCHUNK
    gh[0] = gather(0)
    for c in range(_NCHUNK):
      if c + 1 < _NCHUNK:
        if c >= 1:
          for h in ph[c - 1]:
            h.wait()
        gh[c + 1] = gather(c + 1)
      for h in gh[c]:
        h.wait()
      ph[c] = put(c)
    for h in ph[_NCHUNK - 2]:
      h.wait()
    for h in ph[_NCHUNK - 1]:
      h.wait()

  return emb_gather


_EMB_GATHER = _make_kernel()


def kernel(features, tables):
  B, F = features.shape
  D = tables.shape[-1]
  rg = B // 8
  flat_tables = tables[:, :VOCAB, :].reshape(F * VOCAB, D)
  feati = features.astype(jnp.int32)
  featpad = jnp.concatenate([feati, feati[:, F - _COLS * _FP:]], axis=1)
  vals4 = featpad.reshape(rg, 8, _COLS, _FP).transpose(0, 2, 1, 3)
  f4 = (jnp.arange(_COLS, dtype=jnp.int32)[None, :, None, None] * _FP
        + jnp.arange(_FP, dtype=jnp.int32)[None, None, None, :])
  offs4 = jnp.where(f4 < F, f4 * VOCAB, (f4 - _FP) * VOCAB)
  idx = (vals4 + offs4).reshape(_NW, _NCHUNK, _C128, _FP)
  idx = idx.transpose(0, 1, 3, 2).reshape(-1)
  out = _EMB_GATHER(idx, flat_tables)
  out = out.reshape(rg, _COLS, 8, _LANES).transpose(0, 2, 1, 3)
  return out.reshape(B, _COLS * _LANES)[:, :F * D]


# speedup vs baseline: 1.4123x; 1.0107x over previous
import functools

import jax
import jax.numpy as jnp
from jax import lax
from jax.experimental import pallas as pl
from jax.experimental.pallas import tpu as pltpu
from jax.experimental.pallas import tpu_sc as plsc

NUM_FIELDS = 26
VOCAB = 100000
EMBED_DIM = 16
BATCH = 16384

_NC = 2
_NS = 16
_NW = _NC * _NS

_BF = BATCH * NUM_FIELDS
_B_PER_W = _BF // _NW
_CHUNK = 1664
_NCHUNK = _B_PER_W // _CHUNK


def _make_kernel():
  mesh = plsc.VectorSubcoreMesh(core_axis_name="c", subcore_axis_name="s")

  @functools.partial(
      pl.kernel,
      mesh=mesh,
      out_type=jax.ShapeDtypeStruct((_BF, EMBED_DIM), jnp.float32),
      compiler_params=pltpu.CompilerParams(use_tc_tiling_on_sc=False),
      scratch_types=[
          pltpu.VMEM((_B_PER_W,), jnp.int32),
          pltpu.VMEM((2, _CHUNK, EMBED_DIM), jnp.float32),
          pltpu.SemaphoreType.DMA,
          pltpu.SemaphoreType.DMA,
          pltpu.SemaphoreType.DMA,
          pltpu.SemaphoreType.DMA,
      ],
  )
  def emb_gather(idx_hbm, table_hbm, out_hbm, idx_v, rows_v, g0, g1, s0, s1):
    wid = lax.axis_index("s") * _NC + lax.axis_index("c")
    base = wid * _B_PER_W
    pltpu.sync_copy(idx_hbm.at[pl.ds(base, _B_PER_W)], idx_v)

    gsem = [g0, g1]
    ssem = [s0, s1]

    def gather(c):
      buf = c % 2
      return pltpu.async_copy(
          table_hbm.at[idx_v.at[pl.ds(c * _CHUNK, _CHUNK)]],
          rows_v.at[buf], gsem[buf])

    def put(c):
      buf = c % 2
      return pltpu.async_copy(
          rows_v.at[buf], out_hbm.at[pl.ds(base + c * _CHUNK, _CHUNK)],
          ssem[buf])

    gh = [None] * _NCHUNK
    ph = [None] * _NCHUNK
    gh[0] = gather(0)
    for c in range(_NCHUNK):
      if c + 1 < _NCHUNK:
        if c >= 1:
          ph[c - 1].wait()
        gh[c + 1] = gather(c + 1)
      gh[c].wait()
      ph[c] = put(c)
    ph[_NCHUNK - 2].wait()
    ph[_NCHUNK - 1].wait()

  return emb_gather


_EMB_GATHER = _make_kernel()


def kernel(features, tables):
  B, F = features.shape
  D = tables.shape[-1]
  flat_tables = tables[:, :VOCAB, :].reshape(F * VOCAB, D)
  offsets = (jnp.arange(F, dtype=jnp.int32) * VOCAB)[None, :]
  idx = (features.astype(jnp.int32) + offsets).reshape(-1)
  out = _EMB_GATHER(idx, flat_tables)
  return out.reshape(B, F * D)
